# Initial kernel scaffold; baseline (speedup 1.0000x reference)
import jax
import jax.numpy as jnp
from jax.experimental import pallas as pl


def kernel(x, conv_w0, conv_b0, conv_w1, conv_b1, conv_w2, conv_b2, ln_w0, ln_b0, ln_w1, ln_b1, ln_w2, ln_b2, fc_w, fc_b, ln_w3, ln_b3, q_w, q_b, k_w, k_b, index_sample):
    raise NotImplementedError("write your pallas kernel here")


# speedup vs baseline: 1.0792x; 1.0792x over previous
import jax
import jax.numpy as jnp
import numpy as np
from jax.experimental import pallas as pl


def _layernorm(x, w, b, eps=1e-5):
    mu = x.mean(-1, keepdims=True)
    var = ((x - mu) ** 2).mean(-1, keepdims=True)
    return (x - mu) / jnp.sqrt(var + eps) * w + b


def _conv1d(x, W, b, stride):
    y = jax.lax.conv_general_dilated(x, W, (stride,), 'VALID',
                                     dimension_numbers=('NCH', 'OIH', 'NCH'))
    return y + b[None, :, None]


def _scatter_mean_kernel(mtop_ref, attn_ref, out_ref):
    L = out_ref.shape[1]
    H = attn_ref.shape[1]
    u = attn_ref.shape[2]
    acc = jnp.zeros((L, L), dtype=jnp.float32)
    row_iota = jax.lax.broadcasted_iota(jnp.int32, (L, u), 0)
    for h in range(H):
        mt = mtop_ref[0, h, :]
        at = attn_ref[0, h, :, :]
        at = jnp.where(at < jnp.float32(1.0 / L), jnp.float32(0.0), at)
        oh = (row_iota == mt[None, :]).astype(jnp.float32)
        acc = acc + jax.lax.dot(oh, at, preferred_element_type=jnp.float32)
    out_ref[0] = acc * jnp.float32(1.0 / H)


def _scatter_mean(m_top, attn):
    B, H, u, L = attn.shape
    mtop_f = m_top.astype(jnp.int32)
    return pl.pallas_call(
        _scatter_mean_kernel,
        grid=(B,),
        in_specs=[
            pl.BlockSpec((1, H, u), lambda b: (b, 0, 0)),
            pl.BlockSpec((1, H, u, L), lambda b: (b, 0, 0, 0)),
        ],
        out_specs=pl.BlockSpec((1, L, L), lambda b: (b, 0, 0)),
        out_shape=jax.ShapeDtypeStruct((B, L, L), jnp.float32),
    )(mtop_f, attn)


def kernel(x, conv_w0, conv_b0, conv_w1, conv_b1, conv_w2, conv_b2,
           ln_w0, ln_b0, ln_w1, ln_b1, ln_w2, ln_b2,
           fc_w, fc_b, ln_w3, ln_b3, q_w, q_b, k_w, k_b, index_sample):
    B, N, S = x.shape
    H = 4
    factor = 5
    h = x.reshape(B * N, 1, S)
    layers = [(conv_w0, conv_b0, ln_w0, ln_b0, 2),
              (conv_w1, conv_b1, ln_w1, ln_b1, 2),
              (conv_w2, conv_b2, ln_w2, ln_b2, 2)]
    for (W, b, lw, lb, s) in layers:
        h = _conv1d(h, W, b, s)
        h = jax.nn.relu(h)
        h = _layernorm(h, lw, lb)
    h = h.reshape(B * N, -1)
    h = jax.nn.relu(h @ fc_w + fc_b)
    h = _layernorm(h, ln_w3, ln_b3)
    q = (h @ q_w + q_b).reshape(B, N, H, -1).transpose(0, 2, 1, 3)
    k = (h @ k_w + k_b).reshape(B, N, H, -1).transpose(0, 2, 1, 3)
    L = N
    E = q.shape[-1]
    logL = int(np.ceil(np.log(L)))
    u = min(factor * logL, L)
    K_sample = k[:, :, index_sample, :]
    Q_K_sample = jnp.einsum('bhle,bhlse->bhls', q, K_sample)
    M = Q_K_sample.max(-1) - Q_K_sample.sum(-1) / L
    _, M_top = jax.lax.top_k(M, u)
    Q_reduce = jnp.take_along_axis(q, M_top[..., None], axis=2)
    Q_K = jnp.einsum('bhue,bhle->bhul', Q_reduce, k)
    scale = 1.0 / jnp.sqrt(jnp.asarray(E, dtype=jnp.float32))
    attn = jax.nn.softmax(Q_K * scale, axis=-1)
    return _scatter_mean(M_top, attn)


# speedup vs baseline: 3.9252x; 3.6372x over previous
import jax
import jax.numpy as jnp
import numpy as np
from jax.experimental import pallas as pl


def _layernorm(x, w, b, eps=1e-5):
    mu = x.mean(-1, keepdims=True)
    var = ((x - mu) ** 2).mean(-1, keepdims=True)
    return (x - mu) / jnp.sqrt(var + eps) * w + b


def _conv1d(x, W, b, stride):
    y = jax.lax.conv_general_dilated(x, W, (stride,), 'VALID',
                                     dimension_numbers=('NCH', 'OIH', 'NCH'))
    return y + b[None, :, None]


def _count_kernel(idx_ref, cnt_ref, mask_ref):
    idx = idx_ref[0]
    Rb, U = idx.shape
    Lc = cnt_ref.shape[1]
    colio = jax.lax.broadcasted_iota(jnp.int32, (Rb, Lc), 1)
    cnt = jnp.zeros((Rb, Lc), dtype=jnp.float32)
    for s in range(U):
        cnt = cnt + (colio == idx[:, s:s + 1]).astype(jnp.float32)
    cnt_ref[...] = cnt
    mask_ref[...] = jnp.where(cnt > 0, jnp.float32(0.0), jnp.float32(-3e38))


def _build_count(index_sample, L):
    NB = 8
    Rb = L // NB
    idx3 = index_sample.reshape(NB, Rb, index_sample.shape[1])
    return pl.pallas_call(
        _count_kernel,
        grid=(NB,),
        in_specs=[pl.BlockSpec((1, Rb, idx3.shape[2]), lambda i: (i, 0, 0))],
        out_specs=[pl.BlockSpec((Rb, L), lambda i: (i, 0)),
                   pl.BlockSpec((Rb, L), lambda i: (i, 0))],
        out_shape=[jax.ShapeDtypeStruct((L, L), jnp.float32),
                   jax.ShapeDtypeStruct((L, L), jnp.float32)],
    )(idx3)


def _ck_kernel(cnt_ref, kall_ref, out_ref):
    out_ref[...] = jax.lax.dot(cnt_ref[...], kall_ref[...],
                               preferred_element_type=jnp.float32)


def _matmul_ck(cnt, kall):
    L, D = kall.shape[0], kall.shape[1]
    return pl.pallas_call(
        _ck_kernel,
        out_shape=jax.ShapeDtypeStruct((L, D), jnp.float32),
    )(cnt, kall)


def _score_topk_kernel(q_ref, kT_ref, ck_ref, mask_ref, out_ref, *, u):
    q = q_ref[0]
    kT = kT_ref[0]
    N = q.shape[0]
    qk = jax.lax.dot(q, kT, preferred_element_type=jnp.float32)
    mm = jnp.max(qk + mask_ref[...], axis=1, keepdims=True)
    sums = jnp.sum(q * ck_ref[0], axis=1, keepdims=True)
    M = mm - sums * jnp.float32(1.0 / N)
    iota_col = jax.lax.broadcasted_iota(jnp.int32, (N, 1), 0)
    lane_u = jax.lax.broadcasted_iota(jnp.int32, (1, u), 1)
    mtop = jnp.zeros((1, u), dtype=jnp.int32)
    for i in range(u):
        mx = jnp.max(M)
        idx = jnp.min(jnp.where(M >= mx, iota_col, jnp.int32(N)))
        mtop = mtop + idx * (lane_u == i).astype(jnp.int32)
        M = jnp.where(iota_col == idx, jnp.float32(-jnp.inf), M)
    out_ref[0] = mtop


def _score_topk(qh, kTh, ckh, maskadd, u):
    BH, N, E = qh.shape
    import functools
    return pl.pallas_call(
        functools.partial(_score_topk_kernel, u=u),
        grid=(BH,),
        in_specs=[
            pl.BlockSpec((1, N, E), lambda i: (i, 0, 0)),
            pl.BlockSpec((1, E, N), lambda i: (i, 0, 0)),
            pl.BlockSpec((1, N, E), lambda i: (i, 0, 0)),
            pl.BlockSpec((N, N), lambda i: (0, 0)),
        ],
        out_specs=pl.BlockSpec((1, 1, u), lambda i: (i, 0, 0)),
        out_shape=jax.ShapeDtypeStruct((BH, 1, u), jnp.int32),
    )(qh, kTh, ckh, maskadd)


def _attn_scatter_kernel(mtop_ref, q_ref, kT_ref, out_ref):
    H = q_ref.shape[1]
    N = q_ref.shape[2]
    u = mtop_ref.shape[2]
    acc = jnp.zeros((N, N), dtype=jnp.float32)
    row_iota = jax.lax.broadcasted_iota(jnp.int32, (N, u), 0)
    for h in range(H):
        mt = mtop_ref[0, h:h + 1, :]
        oh = (row_iota == mt).astype(jnp.float32)
        qr = jax.lax.dot_general(oh, q_ref[0, h], (((0,), (0,)), ((), ())),
                                 preferred_element_type=jnp.float32)
        qk2 = jax.lax.dot(qr, kT_ref[0, h],
                          preferred_element_type=jnp.float32) * jnp.float32(0.25)
        sm = qk2 - jnp.max(qk2, axis=1, keepdims=True)
        e = jnp.exp(sm)
        attn = e / jnp.sum(e, axis=1, keepdims=True)
        attn = jnp.where(attn < jnp.float32(1.0 / N), jnp.float32(0.0), attn)
        acc = acc + jax.lax.dot(oh, attn, preferred_element_type=jnp.float32)
    out_ref[0] = acc * jnp.float32(1.0 / H)


def _attn_scatter(mtop, q4, kT4):
    B, H, N, E = q4.shape
    u = mtop.shape[2]
    return pl.pallas_call(
        _attn_scatter_kernel,
        grid=(B,),
        in_specs=[
            pl.BlockSpec((1, H, u), lambda b: (b, 0, 0)),
            pl.BlockSpec((1, H, N, E), lambda b: (b, 0, 0, 0)),
            pl.BlockSpec((1, H, E, N), lambda b: (b, 0, 0, 0)),
        ],
        out_specs=pl.BlockSpec((1, N, N), lambda b: (b, 0, 0)),
        out_shape=jax.ShapeDtypeStruct((B, N, N), jnp.float32),
    )(mtop, q4, kT4)


def kernel(x, conv_w0, conv_b0, conv_w1, conv_b1, conv_w2, conv_b2,
           ln_w0, ln_b0, ln_w1, ln_b1, ln_w2, ln_b2,
           fc_w, fc_b, ln_w3, ln_b3, q_w, q_b, k_w, k_b, index_sample):
    B, N, S = x.shape
    H = 4
    factor = 5
    h = x.reshape(B * N, 1, S)
    layers = [(conv_w0, conv_b0, ln_w0, ln_b0, 2),
              (conv_w1, conv_b1, ln_w1, ln_b1, 2),
              (conv_w2, conv_b2, ln_w2, ln_b2, 2)]
    for (W, b, lw, lb, s) in layers:
        h = _conv1d(h, W, b, s)
        h = jax.nn.relu(h)
        h = _layernorm(h, lw, lb)
    h = h.reshape(B * N, -1)
    h = jax.nn.relu(h @ fc_w + fc_b)
    h = _layernorm(h, ln_w3, ln_b3)
    q_flat = h @ q_w + q_b
    k_flat = h @ k_w + k_b
    L = N
    E = q_flat.shape[1] // H
    logL = int(np.ceil(np.log(L)))
    u = min(factor * logL, L)

    qh = q_flat.reshape(B, N, H, E).transpose(0, 2, 1, 3).reshape(B * H, N, E)
    kTh = k_flat.reshape(B, N, H, E).transpose(0, 2, 3, 1).reshape(B * H, E, N)
    kall = k_flat.reshape(B, N, H * E).transpose(1, 0, 2).reshape(N, B * H * E)

    cnt, maskadd = _build_count(index_sample, L)
    ck = _matmul_ck(cnt, kall)
    ckh = ck.reshape(N, B, H, E).transpose(1, 2, 0, 3).reshape(B * H, N, E)

    mtop = _score_topk(qh, kTh, ckh, maskadd, u)
    mtop = mtop.reshape(B, H, u)
    return _attn_scatter(mtop, qh.reshape(B, H, N, E), kTh.reshape(B, H, E, N))


# speedup vs baseline: 6.0118x; 1.5316x over previous
import jax
import jax.numpy as jnp
import numpy as np
from jax.experimental import pallas as pl


def _layernorm(x, w, b, eps=1e-5):
    mu = x.mean(-1, keepdims=True)
    var = ((x - mu) ** 2).mean(-1, keepdims=True)
    return (x - mu) / jnp.sqrt(var + eps) * w + b


def _conv1d(x, W, b, stride):
    y = jax.lax.conv_general_dilated(x, W, (stride,), 'VALID',
                                     dimension_numbers=('NCH', 'OIH', 'NCH'))
    return y + b[None, :, None]


def _count_kernel(idxT_ref, cntT_ref, maskT_ref):
    idxT = idxT_ref[...]
    U, Lc = idxT.shape
    Rb = cntT_ref.shape[0]
    j0 = pl.program_id(0) * Rb
    rowio = jax.lax.broadcasted_iota(jnp.int32, (Rb, Lc), 0) + j0
    cnt = jnp.zeros((Rb, Lc), dtype=jnp.float32)
    for s in range(U):
        cnt = cnt + (rowio == idxT[s:s + 1, :]).astype(jnp.float32)
    cntT_ref[...] = cnt
    maskT_ref[...] = jnp.where(cnt > 0, jnp.float32(0.0), jnp.float32(-3e38))


def _build_count(idxT, L):
    NB = 8
    Rb = L // NB
    return pl.pallas_call(
        _count_kernel,
        grid=(NB,),
        in_specs=[pl.BlockSpec((idxT.shape[0], L), lambda i: (0, 0))],
        out_specs=[pl.BlockSpec((Rb, L), lambda i: (i, 0)),
                   pl.BlockSpec((Rb, L), lambda i: (i, 0))],
        out_shape=[jax.ShapeDtypeStruct((L, L), jnp.float32),
                   jax.ShapeDtypeStruct((L, L), jnp.float32)],
    )(idxT)


def _ck_kernel(cntT_ref, kall_ref, out_ref):
    out_ref[...] = jax.lax.dot_general(
        cntT_ref[...], kall_ref[...], (((0,), (0,)), ((), ())),
        preferred_element_type=jnp.float32)


def _matmul_ck(cnt, kall):
    L, D = kall.shape[0], kall.shape[1]
    return pl.pallas_call(
        _ck_kernel,
        out_shape=jax.ShapeDtypeStruct((L, D), jnp.float32),
    )(cnt, kall)


def _score_topk_kernel(qT_ref, kT_ref, ckT_ref, maskT_ref, out_ref, *, u):
    BH = qT_ref.shape[0]
    N = qT_ref.shape[2]
    maskT = maskT_ref[...]
    rows = []
    for bh in range(BH):
        qT = qT_ref[bh]
        kT = kT_ref[bh]
        qkT = jax.lax.dot_general(kT, qT, (((0,), (0,)), ((), ())),
                                  preferred_element_type=jnp.float32)
        mm = jnp.max(qkT + maskT, axis=0, keepdims=True)
        sums = jnp.sum(qT * ckT_ref[bh], axis=0, keepdims=True)
        rows.append(mm - sums * jnp.float32(1.0 / N))
    M = jnp.concatenate(rows, axis=0)
    iota_row = jax.lax.broadcasted_iota(jnp.int32, (BH, N), 1)
    lane_u = jax.lax.broadcasted_iota(jnp.int32, (1, u), 1)
    mtop = jnp.zeros((BH, u), dtype=jnp.int32)
    for i in range(u):
        mx = jnp.max(M, axis=1, keepdims=True)
        idx = jnp.min(jnp.where(M >= mx, iota_row, jnp.int32(N)),
                      axis=1, keepdims=True)
        mtop = mtop + idx * (lane_u == i).astype(jnp.int32)
        M = jnp.where(iota_row == idx, jnp.float32(-jnp.inf), M)
    out_ref[...] = mtop


def _score_topk(qTh, kTh, ckTh, maskT, u):
    BH, E, N = qTh.shape
    import functools
    return pl.pallas_call(
        functools.partial(_score_topk_kernel, u=u),
        out_shape=jax.ShapeDtypeStruct((BH, u), jnp.int32),
    )(qTh, kTh, ckTh, maskT)


def _attn_scatter_kernel(mtop_ref, q_ref, kT_ref, out_ref):
    H = q_ref.shape[1]
    N = q_ref.shape[2]
    u = mtop_ref.shape[2]
    acc = jnp.zeros((N, N), dtype=jnp.float32)
    row_iota = jax.lax.broadcasted_iota(jnp.int32, (N, u), 0)
    for h in range(H):
        mt = mtop_ref[0, h:h + 1, :]
        oh = (row_iota == mt).astype(jnp.float32)
        qr = jax.lax.dot_general(oh, q_ref[0, h], (((0,), (0,)), ((), ())),
                                 preferred_element_type=jnp.float32)
        qk2 = jax.lax.dot(qr, kT_ref[0, h],
                          preferred_element_type=jnp.float32) * jnp.float32(0.25)
        sm = qk2 - jnp.max(qk2, axis=1, keepdims=True)
        e = jnp.exp(sm)
        attn = e / jnp.sum(e, axis=1, keepdims=True)
        attn = jnp.where(attn < jnp.float32(1.0 / N), jnp.float32(0.0), attn)
        acc = acc + jax.lax.dot(oh, attn, preferred_element_type=jnp.float32)
    out_ref[0] = acc * jnp.float32(1.0 / H)


def _attn_scatter(mtop, q4, kT4):
    B, H, N, E = q4.shape
    u = mtop.shape[2]
    return pl.pallas_call(
        _attn_scatter_kernel,
        grid=(B,),
        in_specs=[
            pl.BlockSpec((1, H, u), lambda b: (b, 0, 0)),
            pl.BlockSpec((1, H, N, E), lambda b: (b, 0, 0, 0)),
            pl.BlockSpec((1, H, E, N), lambda b: (b, 0, 0, 0)),
        ],
        out_specs=pl.BlockSpec((1, N, N), lambda b: (b, 0, 0)),
        out_shape=jax.ShapeDtypeStruct((B, N, N), jnp.float32),
    )(mtop, q4, kT4)


def kernel(x, conv_w0, conv_b0, conv_w1, conv_b1, conv_w2, conv_b2,
           ln_w0, ln_b0, ln_w1, ln_b1, ln_w2, ln_b2,
           fc_w, fc_b, ln_w3, ln_b3, q_w, q_b, k_w, k_b, index_sample):
    B, N, S = x.shape
    H = 4
    factor = 5
    h = x.reshape(B * N, 1, S)
    layers = [(conv_w0, conv_b0, ln_w0, ln_b0, 2),
              (conv_w1, conv_b1, ln_w1, ln_b1, 2),
              (conv_w2, conv_b2, ln_w2, ln_b2, 2)]
    for (W, b, lw, lb, s) in layers:
        h = _conv1d(h, W, b, s)
        h = jax.nn.relu(h)
        h = _layernorm(h, lw, lb)
    h = h.reshape(B * N, -1)
    h = jax.nn.relu(h @ fc_w + fc_b)
    h = _layernorm(h, ln_w3, ln_b3)
    q_flat = h @ q_w + q_b
    k_flat = h @ k_w + k_b
    L = N
    E = q_flat.shape[1] // H
    logL = int(np.ceil(np.log(L)))
    u = min(factor * logL, L)

    qTh = q_flat.reshape(B, N, H, E).transpose(0, 2, 3, 1).reshape(B * H, E, N)
    kTh = k_flat.reshape(B, N, H, E).transpose(0, 2, 3, 1).reshape(B * H, E, N)
    kall = k_flat.reshape(B, N, H * E).transpose(1, 0, 2).reshape(N, B * H * E)

    cntT, maskT = _build_count(index_sample.T, L)
    ck = _matmul_ck(cntT, kall)
    ckTh = ck.reshape(N, B, H, E).transpose(1, 2, 3, 0).reshape(B * H, E, N)

    mtop = _score_topk(qTh, kTh, ckTh, maskT, u)
    mtop = mtop.reshape(B, H, u)
    q4 = q_flat.reshape(B, N, H, E).transpose(0, 2, 1, 3)
    return _attn_scatter(mtop, q4, kTh.reshape(B, H, E, N))
